# Initial kernel scaffold; baseline (speedup 1.0000x reference)
import jax
import jax.numpy as jnp
from jax.experimental import pallas as pl


def kernel(x, edge_index, e, i, idx_a, idx_b, W_e1, b_e1, W_root1, b1, W_e2, b_e2, W_root2, b2, W_d, b_d):
    raise NotImplementedError("write your pallas kernel here")


# speedup vs baseline: 1.9199x; 1.9199x over previous
import functools

import jax
import jax.numpy as jnp
from jax import lax
from jax.experimental import pallas as pl
from jax.experimental.pallas import tpu as pltpu
from jax.experimental.pallas import tpu_sc as plsc

_N = 10000
_E = 160000
_F = 128
_H = 32
_DE = 16
_P = 8192

_NC = 2
_NS = 16
_NW = _NC * _NS
_EPW = _E // _NW
_C = 40
_CH = _EPW // _C
_NA = 10240
_NPS = _NA // _NS
_ZB = 128
_PPW = _P // _NW

_mesh = plsc.VectorSubcoreMesh(core_axis_name="c", subcore_axis_name="s")


def _worker_id():
    return lax.axis_index("s") * _NC + lax.axis_index("c")


def _make_sc_gather(d_cols):
    @functools.partial(
        pl.kernel,
        out_type=jax.ShapeDtypeStruct((_E, d_cols), jnp.float32),
        mesh=_mesh,
        scratch_types=[
            pltpu.VMEM((_CH, _C), jnp.int32),
            pltpu.VMEM((_C, d_cols), jnp.float32),
            pltpu.SemaphoreType.DMA,
        ],
    )
    def sc_gather(table_hbm, idx_hbm, out_hbm, idxv, rowsv, sem):
        w = _worker_id()
        pltpu.sync_copy(idx_hbm.at[w], idxv)

        def chunk(j, carry):
            pltpu.async_copy(table_hbm.at[idxv.at[j]], rowsv, sem).wait()
            off = pl.multiple_of(w * _EPW + j * _C, 8)
            pltpu.sync_copy(rowsv, out_hbm.at[pl.ds(off, _C)])
            return carry

        lax.fori_loop(0, _CH, chunk, 0)

    return sc_gather


@functools.partial(
    pl.kernel,
    out_type=jax.ShapeDtypeStruct((_NC, _NA, _F), jnp.float32),
    mesh=_mesh,
    scratch_types=[
        pltpu.VMEM((_C,), jnp.int32),
        pltpu.VMEM((_C, _F), jnp.float32),
        pltpu.VMEM((_ZB, _F), jnp.float32),
        pltpu.VMEM_SHARED((_NA, _F), jnp.float32),
        pltpu.SemaphoreType.DMA,
    ],
)
def _sc_scatter_add(msgs_hbm, idx_hbm, out_hbm, idxv, mv, zbuf, aggsh, sem):
    c = lax.axis_index("c")
    s = lax.axis_index("s")
    w = _worker_id()

    def zfill(i, carry):
        for q in range(_F // 16):
            zbuf[i, pl.ds(q * 16, 16)] = jnp.zeros((16,), jnp.float32)
        return carry

    lax.fori_loop(0, _ZB, zfill, 0)
    for k in range(_NPS // _ZB):
        zoff = pl.multiple_of(s * _NPS + k * _ZB, 8)
        pltpu.sync_copy(zbuf, aggsh.at[pl.ds(zoff, _ZB)])
    plsc.subcore_barrier()

    def chunk(j, carry):
        off = pl.multiple_of(w * _EPW + j * _C, 8)
        pltpu.sync_copy(idx_hbm.at[pl.ds(off, _C)], idxv)
        pltpu.sync_copy(msgs_hbm.at[pl.ds(off, _C)], mv)
        pltpu.sync_copy(mv, aggsh.at[idxv], add=True)
        return carry

    lax.fori_loop(0, _CH, chunk, 0)
    plsc.subcore_barrier()
    doff = pl.multiple_of(s * _NPS, 8)
    pltpu.sync_copy(aggsh.at[pl.ds(doff, _NPS)],
                    out_hbm.at[c, pl.ds(doff, _NPS)])


@functools.partial(
    pl.kernel,
    out_type=jax.ShapeDtypeStruct((_P,), jnp.float32),
    mesh=_mesh,
    scratch_types=[
        pltpu.VMEM((_N,), jnp.float32),
        pltpu.VMEM((_PPW,), jnp.int32),
        pltpu.VMEM((_PPW,), jnp.int32),
        pltpu.VMEM((_PPW,), jnp.float32),
    ],
    compiler_params=pltpu.CompilerParams(needs_layout_passes=False),
)
def _sc_pref(util_hbm, ia_hbm, ib_hbm, out_hbm, utilv, iav, ibv, outv):
    w = _worker_id()
    base = pl.multiple_of(w * _PPW, 8)
    pltpu.sync_copy(util_hbm, utilv)
    pltpu.sync_copy(ia_hbm.at[pl.ds(base, _PPW)], iav)
    pltpu.sync_copy(ib_hbm.at[pl.ds(base, _PPW)], ibv)

    def body(j, carry):
        ia = iav[pl.ds(j * 16, 16)]
        ib = ibv[pl.ds(j * 16, 16)]
        va = plsc.load_gather(utilv, [ia])
        vb = plsc.load_gather(utilv, [ib])
        outv[pl.ds(j * 16, 16)] = vb - va
        return carry

    lax.fori_loop(0, _PPW // 16, body, 0)
    pltpu.sync_copy(outv, out_hbm.at[pl.ds(base, _PPW)])


def _tc_combine(xs, e, w_cat, f_in, block):
    def body(xs_ref, e_ref, w_ref, o_ref):
        g = jnp.dot(xs_ref[...], w_ref[...], preferred_element_type=jnp.float32)
        eb = e_ref[...]
        acc = g[:, _DE * _H:(_DE + 1) * _H]
        for d in range(_DE):
            acc = acc + eb[:, d:d + 1] * g[:, d * _H:(d + 1) * _H]
        o_ref[:, :_H] = acc
        o_ref[:, _H:] = jnp.zeros((block, _F - _H), jnp.float32)

    nblk = _E // block
    return pl.pallas_call(
        body,
        grid=(nblk,),
        in_specs=[
            pl.BlockSpec((block, f_in), lambda i: (i, 0)),
            pl.BlockSpec((block, _DE), lambda i: (i, 0)),
            pl.BlockSpec((f_in, (_DE + 1) * _H), lambda i: (0, 0)),
        ],
        out_specs=pl.BlockSpec((block, _F), lambda i: (i, 0)),
        out_shape=jax.ShapeDtypeStruct((_E, _F), jnp.float32),
    )(xs, e, w_cat)


def _tc_relu_root(agg, x, w_root, b, f_in, block):
    def body(agg_ref, x_ref, w_ref, b_ref, o_ref):
        a = (agg_ref[0, :, :_H] + agg_ref[1, :, :_H]
             + jnp.dot(x_ref[...], w_ref[...], preferred_element_type=jnp.float32)
             + b_ref[...])
        o_ref[:, :_H] = jnp.maximum(a, 0.0)
        o_ref[:, _H:] = jnp.zeros((block, _F - _H), jnp.float32)

    nblk = _N // block
    return pl.pallas_call(
        body,
        grid=(nblk,),
        in_specs=[
            pl.BlockSpec((2, block, _F), lambda i: (0, i, 0)),
            pl.BlockSpec((block, f_in), lambda i: (i, 0)),
            pl.BlockSpec((f_in, _H), lambda i: (0, 0)),
            pl.BlockSpec((1, _H), lambda i: (0, 0)),
        ],
        out_specs=pl.BlockSpec((block, _F), lambda i: (i, 0)),
        out_shape=jax.ShapeDtypeStruct((_N, _F), jnp.float32),
    )(agg, x, w_root, b)


def _tc_final(agg, x1, w_root, b, wd_row, bd, block):
    def body(agg_ref, x_ref, w_ref, b_ref, wd_ref, bd_ref, o_ref):
        a = (agg_ref[0, :, :_H] + agg_ref[1, :, :_H]
             + jnp.dot(x_ref[...], w_ref[...], preferred_element_type=jnp.float32)
             + b_ref[...])
        x2 = jnp.maximum(a, 0.0)
        o_ref[...] = jnp.sum(x2 * wd_ref[...], axis=1, keepdims=True) + bd_ref[...]

    nblk = _N // block
    return pl.pallas_call(
        body,
        grid=(nblk,),
        in_specs=[
            pl.BlockSpec((2, block, _F), lambda i: (0, i, 0)),
            pl.BlockSpec((block, _F), lambda i: (i, 0)),
            pl.BlockSpec((_F, _H), lambda i: (0, 0)),
            pl.BlockSpec((1, _H), lambda i: (0, 0)),
            pl.BlockSpec((1, _H), lambda i: (0, 0)),
            pl.BlockSpec((1, 1), lambda i: (0, 0)),
        ],
        out_specs=pl.BlockSpec((block, 1), lambda i: (i, 0)),
        out_shape=jax.ShapeDtypeStruct((_N, 1), jnp.float32),
    )(agg, x1, w_root, b, wd_row, bd)


_gather_f = _make_sc_gather(_F)


def kernel(x, edge_index, e, i, idx_a, idx_b, W_e1, b_e1, W_root1, b1,
           W_e2, b_e2, W_root2, b2, W_d, b_d):
    del i
    x = x.astype(jnp.float32)
    src = edge_index[0].reshape(_NW, _CH, _C)

    w1 = W_e1.reshape(_DE, _F, _H).transpose(1, 0, 2).reshape(_F, _DE * _H)
    w1_cat = jnp.concatenate([w1, b_e1.reshape(_F, _H)], axis=1)
    w2 = W_e2.reshape(_DE, _H, _H).transpose(1, 0, 2).reshape(_H, _DE * _H)
    w2_cat = jnp.pad(jnp.concatenate([w2, b_e2.reshape(_H, _H)], axis=1),
                     ((0, _F - _H), (0, 0)))
    w_root2 = jnp.pad(W_root2, ((0, _F - _H), (0, 0)))

    dst_flat = edge_index[1]
    xs1 = _gather_f(x, src)
    msgs1 = _tc_combine(xs1, e, w1_cat, _F, 2000)
    agg1 = _sc_scatter_add(msgs1, dst_flat)
    x1 = _tc_relu_root(agg1, x, W_root1, b1.reshape(1, _H), _F, 2000)

    xs2 = _gather_f(x1, src)
    msgs2 = _tc_combine(xs2, e, w2_cat, _F, 2000)
    agg2 = _sc_scatter_add(msgs2, dst_flat)
    util = _tc_final(agg2, x1, w_root2, b2.reshape(1, _H),
                     W_d.reshape(1, _H), b_d.reshape(1, 1), 2000)

    out = _sc_pref(util.reshape(_N), idx_a, idx_b)
    return out.reshape(_P, 1)


# speedup vs baseline: 2.3748x; 1.2370x over previous
import functools

import jax
import jax.numpy as jnp
from jax import lax
from jax.experimental import pallas as pl
from jax.experimental.pallas import tpu as pltpu
from jax.experimental.pallas import tpu_sc as plsc

_N = 10000
_E = 160000
_F = 128
_H = 32
_DE = 16
_P = 8192

_NC = 2
_NS = 16
_NW = _NC * _NS
_C = 128
_CH = 40
_EPW = _CH * _C
_EPAD = _NW * _EPW
_NA = 10240
_NPS = _NA // _NS
_ZB = 64
_PPW = _P // _NW
_NB = 2

_mesh = plsc.VectorSubcoreMesh(core_axis_name="c", subcore_axis_name="s")


def _worker_id():
    return lax.axis_index("s") * _NC + lax.axis_index("c")


@functools.partial(
    pl.kernel,
    out_type=jax.ShapeDtypeStruct((_EPAD, _F), jnp.float32),
    mesh=_mesh,
    scratch_types=(
        [pltpu.VMEM((_CH, _C), jnp.int32),
         pltpu.VMEM_SHARED((_NA, _F), jnp.float32)]
        + [pltpu.VMEM((_C, _F), jnp.float32) for _ in range(_NB)]
        + [pltpu.SemaphoreType.DMA for _ in range(2 * _NB)]
    ),
)
def _sc_gather(table_hbm, idx_hbm, out_hbm, idxv, tsh,
               r0, r1, g0, g1, q0, q1):
    s = lax.axis_index("s")
    w = _worker_id()
    bufs = [r0, r1]
    gsems = [g0, g1]
    wsems = [q0, q1]

    stripe = _NA // _NS
    soff = pl.multiple_of(s * stripe, 8)

    @pl.when(s < _NS - 1)
    def _():
        pltpu.sync_copy(table_hbm.at[pl.ds(soff, stripe)],
                        tsh.at[pl.ds(soff, stripe)])

    @pl.when(s == _NS - 1)
    def _():
        last = _N - (_NS - 1) * stripe
        loff = pl.multiple_of((_NS - 1) * stripe, 8)
        pltpu.sync_copy(table_hbm.at[pl.ds(loff, last)],
                        tsh.at[pl.ds(loff, last)])

    pltpu.sync_copy(idx_hbm.at[w], idxv)
    plsc.subcore_barrier()

    g = [None] * _NB
    wr = [None] * _NB
    for j in range(min(_NB - 1, _CH)):
        g[j] = pltpu.async_copy(tsh.at[idxv.at[j]], bufs[j], gsems[j])
    for j in range(_CH):
        b = j % _NB
        g[b].wait()
        off = pl.multiple_of(w * _EPW + j * _C, 8)
        wr[b] = pltpu.async_copy(bufs[b], out_hbm.at[pl.ds(off, _C)], wsems[b])
        nx = j + _NB - 1
        if nx < _CH:
            nb = nx % _NB
            if wr[nb] is not None:
                wr[nb].wait()
            g[nb] = pltpu.async_copy(tsh.at[idxv.at[nx]], bufs[nb], gsems[nb])
    for b in range(_NB):
        if wr[b] is not None:
            wr[b].wait()


@functools.partial(
    pl.kernel,
    out_type=jax.ShapeDtypeStruct((_NC, _NA, _F), jnp.float32),
    mesh=_mesh,
    scratch_types=(
        [pltpu.VMEM((_ZB, _F), jnp.float32),
         pltpu.VMEM_SHARED((_NA, _F), jnp.float32)]
        + [pltpu.VMEM((_C,), jnp.int32) for _ in range(_NB)]
        + [pltpu.VMEM((_C, _F), jnp.float32) for _ in range(_NB)]
        + [pltpu.SemaphoreType.DMA for _ in range(2 * _NB)]
    ),
)
def _sc_scatter_add(msgs_hbm, idx_hbm, out_hbm, zbuf, aggsh,
                    i0, i1, m0, m1, s0, s1, t0, t1):
    c = lax.axis_index("c")
    s = lax.axis_index("s")
    w = _worker_id()
    ibufs = [i0, i1]
    mbufs = [m0, m1]
    isems = [s0, s1]
    msems = [t0, t1]

    def zfill(i, carry):
        for q in range(_F // 16):
            zbuf[i, pl.ds(q * 16, 16)] = jnp.zeros((16,), jnp.float32)
        return carry

    lax.fori_loop(0, _ZB, zfill, 0)
    for k in range(_NPS // _ZB):
        zoff = pl.multiple_of(s * _NPS + k * _ZB, 8)
        pltpu.sync_copy(zbuf, aggsh.at[pl.ds(zoff, _ZB)])
    plsc.subcore_barrier()

    il = [None] * _NB
    ml = [None] * _NB

    def start_loads(j):
        b = j % _NB
        off = pl.multiple_of(w * _EPW + j * _C, 8)
        il[b] = pltpu.async_copy(idx_hbm.at[pl.ds(off, _C)], ibufs[b], isems[b])
        ml[b] = pltpu.async_copy(msgs_hbm.at[pl.ds(off, _C)], mbufs[b], msems[b])

    for j in range(min(_NB - 1, _CH)):
        start_loads(j)
    for j in range(_CH):
        b = j % _NB
        il[b].wait()
        ml[b].wait()
        pltpu.sync_copy(mbufs[b], aggsh.at[ibufs[b]], add=True)
        if j + _NB - 1 < _CH:
            start_loads(j + _NB - 1)

    plsc.subcore_barrier()
    doff = pl.multiple_of(s * _NPS, 8)
    pltpu.sync_copy(aggsh.at[pl.ds(doff, _NPS)],
                    out_hbm.at[c, pl.ds(doff, _NPS)])


@functools.partial(
    pl.kernel,
    out_type=jax.ShapeDtypeStruct((_P,), jnp.float32),
    mesh=_mesh,
    scratch_types=[
        pltpu.VMEM((_N,), jnp.float32),
        pltpu.VMEM((_PPW,), jnp.int32),
        pltpu.VMEM((_PPW,), jnp.int32),
        pltpu.VMEM((_PPW,), jnp.float32),
    ],
    compiler_params=pltpu.CompilerParams(needs_layout_passes=False),
)
def _sc_pref(util_hbm, ia_hbm, ib_hbm, out_hbm, utilv, iav, ibv, outv):
    w = _worker_id()
    base = pl.multiple_of(w * _PPW, 8)
    pltpu.sync_copy(util_hbm, utilv)
    pltpu.sync_copy(ia_hbm.at[pl.ds(base, _PPW)], iav)
    pltpu.sync_copy(ib_hbm.at[pl.ds(base, _PPW)], ibv)

    def body(j, carry):
        ia = iav[pl.ds(j * 16, 16)]
        ib = ibv[pl.ds(j * 16, 16)]
        va = plsc.load_gather(utilv, [ia])
        vb = plsc.load_gather(utilv, [ib])
        outv[pl.ds(j * 16, 16)] = vb - va
        return carry

    lax.fori_loop(0, _PPW // 16, body, 0)
    pltpu.sync_copy(outv, out_hbm.at[pl.ds(base, _PPW)])


def _tc_combine(xs, e, w_cat, f_in, block):
    def body(xs_ref, e_ref, w_ref, o_ref):
        g = jnp.dot(xs_ref[...], w_ref[...], preferred_element_type=jnp.float32)
        eb = e_ref[...]
        acc = g[:, _DE * _H:(_DE + 1) * _H]
        for d in range(_DE):
            acc = acc + eb[:, d:d + 1] * g[:, d * _H:(d + 1) * _H]
        o_ref[:, :_H] = acc
        o_ref[:, _H:] = jnp.zeros((block, _F - _H), jnp.float32)

    nblk = _EPAD // block
    return pl.pallas_call(
        body,
        grid=(nblk,),
        in_specs=[
            pl.BlockSpec((block, f_in), lambda i: (i, 0)),
            pl.BlockSpec((block, _DE), lambda i: (i, 0)),
            pl.BlockSpec((f_in, (_DE + 1) * _H), lambda i: (0, 0)),
        ],
        out_specs=pl.BlockSpec((block, _F), lambda i: (i, 0)),
        out_shape=jax.ShapeDtypeStruct((_EPAD, _F), jnp.float32),
    )(xs, e, w_cat)


def _tc_relu_root(agg, x, w_root, b, f_in, block):
    def body(agg_ref, x_ref, w_ref, b_ref, o_ref):
        a = (agg_ref[0, :, :_H] + agg_ref[1, :, :_H]
             + jnp.dot(x_ref[...], w_ref[...], preferred_element_type=jnp.float32)
             + b_ref[...])
        o_ref[:, :_H] = jnp.maximum(a, 0.0)
        o_ref[:, _H:] = jnp.zeros((block, _F - _H), jnp.float32)

    nblk = _N // block
    return pl.pallas_call(
        body,
        grid=(nblk,),
        in_specs=[
            pl.BlockSpec((2, block, _F), lambda i: (0, i, 0)),
            pl.BlockSpec((block, f_in), lambda i: (i, 0)),
            pl.BlockSpec((f_in, _H), lambda i: (0, 0)),
            pl.BlockSpec((1, _H), lambda i: (0, 0)),
        ],
        out_specs=pl.BlockSpec((block, _F), lambda i: (i, 0)),
        out_shape=jax.ShapeDtypeStruct((_N, _F), jnp.float32),
    )(agg, x, w_root, b)


def _tc_final(agg, x1, w_root, b, wd_row, bd, block):
    def body(agg_ref, x_ref, w_ref, b_ref, wd_ref, bd_ref, o_ref):
        a = (agg_ref[0, :, :_H] + agg_ref[1, :, :_H]
             + jnp.dot(x_ref[...], w_ref[...], preferred_element_type=jnp.float32)
             + b_ref[...])
        x2 = jnp.maximum(a, 0.0)
        o_ref[...] = jnp.sum(x2 * wd_ref[...], axis=1, keepdims=True) + bd_ref[...]

    nblk = _N // block
    return pl.pallas_call(
        body,
        grid=(nblk,),
        in_specs=[
            pl.BlockSpec((2, block, _F), lambda i: (0, i, 0)),
            pl.BlockSpec((block, _F), lambda i: (i, 0)),
            pl.BlockSpec((_F, _H), lambda i: (0, 0)),
            pl.BlockSpec((1, _H), lambda i: (0, 0)),
            pl.BlockSpec((1, _H), lambda i: (0, 0)),
            pl.BlockSpec((1, 1), lambda i: (0, 0)),
        ],
        out_specs=pl.BlockSpec((block, 1), lambda i: (i, 0)),
        out_shape=jax.ShapeDtypeStruct((_N, 1), jnp.float32),
    )(agg, x1, w_root, b, wd_row, bd)


def kernel(x, edge_index, e, i, idx_a, idx_b, W_e1, b_e1, W_root1, b1,
           W_e2, b_e2, W_root2, b2, W_d, b_d):
    del i
    x = x.astype(jnp.float32)
    npad = _EPAD - _E
    src = jnp.concatenate(
        [edge_index[0], jnp.zeros((npad,), jnp.int32)]).reshape(_NW, _CH, _C)
    dst_flat = jnp.concatenate(
        [edge_index[1], jnp.full((npad,), _N, jnp.int32)])
    e_p = jnp.concatenate([e, jnp.zeros((npad, _DE), jnp.float32)])

    w1 = W_e1.reshape(_DE, _F, _H).transpose(1, 0, 2).reshape(_F, _DE * _H)
    w1_cat = jnp.concatenate([w1, b_e1.reshape(_F, _H)], axis=1)
    w2 = W_e2.reshape(_DE, _H, _H).transpose(1, 0, 2).reshape(_H, _DE * _H)
    w2_cat = jnp.pad(jnp.concatenate([w2, b_e2.reshape(_H, _H)], axis=1),
                     ((0, _F - _H), (0, 0)))
    w_root2 = jnp.pad(W_root2, ((0, _F - _H), (0, 0)))

    xs1 = _sc_gather(x, src)
    msgs1 = _tc_combine(xs1, e_p, w1_cat, _F, 2048)
    agg1 = _sc_scatter_add(msgs1, dst_flat)
    x1 = _tc_relu_root(agg1, x, W_root1, b1.reshape(1, _H), _F, 2000)

    xs2 = _sc_gather(x1, src)
    msgs2 = _tc_combine(xs2, e_p, w2_cat, _F, 2048)
    agg2 = _sc_scatter_add(msgs2, dst_flat)
    util = _tc_final(agg2, x1, w_root2, b2.reshape(1, _H),
                     W_d.reshape(1, _H), b_d.reshape(1, 1), 2000)

    out = _sc_pref(util.reshape(_N), idx_a, idx_b)
    return out.reshape(_P, 1)


# speedup vs baseline: 5.1477x; 2.1676x over previous
import functools

import jax
import jax.numpy as jnp
from jax import lax
from jax.experimental import pallas as pl
from jax.experimental.pallas import tpu as pltpu
from jax.experimental.pallas import tpu_sc as plsc

_N = 10000
_E = 160000
_F = 128
_H = 32
_DE = 16
_P = 8192

_NC = 2
_NS = 16
_NW = _NC * _NS
_C = 128
_CH = 40
_EPW = _CH * _C
_EPAD = _NW * _EPW
_NA = 10240
_NPS = _NA // _NS
_ZB = 64
_PPW = _P // _NW
_NB = 2

_mesh = plsc.VectorSubcoreMesh(core_axis_name="c", subcore_axis_name="s")


def _worker_id():
    return lax.axis_index("s") * _NC + lax.axis_index("c")


@functools.partial(
    pl.kernel,
    out_type=jax.ShapeDtypeStruct((_EPAD, _F), jnp.float32),
    mesh=_mesh,
    scratch_types=(
        [pltpu.VMEM((_CH, _C), jnp.int32),
         pltpu.VMEM_SHARED((_NA, _F), jnp.float32)]
        + [pltpu.VMEM((_C, _F), jnp.float32) for _ in range(_NB)]
        + [pltpu.SemaphoreType.DMA for _ in range(2 * _NB)]
    ),
)
def _sc_gather(table_hbm, idx_hbm, out_hbm, idxv, tsh,
               r0, r1, g0, g1, q0, q1):
    s = lax.axis_index("s")
    w = _worker_id()
    bufs = [r0, r1]
    gsems = [g0, g1]
    wsems = [q0, q1]

    stripe = _NA // _NS
    soff = pl.multiple_of(s * stripe, 8)

    @pl.when(s < _NS - 1)
    def _():
        pltpu.sync_copy(table_hbm.at[pl.ds(soff, stripe)],
                        tsh.at[pl.ds(soff, stripe)])

    @pl.when(s == _NS - 1)
    def _():
        last = _N - (_NS - 1) * stripe
        loff = pl.multiple_of((_NS - 1) * stripe, 8)
        pltpu.sync_copy(table_hbm.at[pl.ds(loff, last)],
                        tsh.at[pl.ds(loff, last)])

    pltpu.sync_copy(idx_hbm.at[w], idxv)
    plsc.subcore_barrier()

    g = [None] * _NB
    wr = [None] * _NB
    for j in range(min(_NB - 1, _CH)):
        g[j] = pltpu.async_copy(tsh.at[idxv.at[j]], bufs[j], gsems[j])
    for j in range(_CH):
        b = j % _NB
        g[b].wait()
        off = pl.multiple_of(w * _EPW + j * _C, 8)
        wr[b] = pltpu.async_copy(bufs[b], out_hbm.at[pl.ds(off, _C)], wsems[b])
        nx = j + _NB - 1
        if nx < _CH:
            nb = nx % _NB
            if wr[nb] is not None:
                wr[nb].wait()
            g[nb] = pltpu.async_copy(tsh.at[idxv.at[nx]], bufs[nb], gsems[nb])
    for b in range(_NB):
        if wr[b] is not None:
            wr[b].wait()


@functools.partial(
    pl.kernel,
    out_type=jax.ShapeDtypeStruct((_NC, _NA, _F), jnp.float32),
    mesh=_mesh,
    scratch_types=(
        [pltpu.VMEM((_ZB, _F), jnp.float32),
         pltpu.VMEM_SHARED((_NA, _F), jnp.float32)]
        + [pltpu.VMEM((_C,), jnp.int32) for _ in range(_NB)]
        + [pltpu.VMEM((_C, _F), jnp.float32) for _ in range(_NB)]
        + [pltpu.SemaphoreType.DMA for _ in range(2 * _NB)]
    ),
)
def _sc_scatter_add(msgs_hbm, idx_hbm, out_hbm, zbuf, aggsh,
                    i0, i1, m0, m1, s0, s1, t0, t1):
    c = lax.axis_index("c")
    s = lax.axis_index("s")
    w = _worker_id()
    ibufs = [i0, i1]
    mbufs = [m0, m1]
    isems = [s0, s1]
    msems = [t0, t1]

    def zfill(i, carry):
        for q in range(_F // 16):
            zbuf[i, pl.ds(q * 16, 16)] = jnp.zeros((16,), jnp.float32)
        return carry

    lax.fori_loop(0, _ZB, zfill, 0)
    for k in range(_NPS // _ZB):
        zoff = pl.multiple_of(s * _NPS + k * _ZB, 8)
        pltpu.sync_copy(zbuf, aggsh.at[pl.ds(zoff, _ZB)])
    plsc.subcore_barrier()

    il = [None] * _NB
    ml = [None] * _NB

    def start_loads(j):
        b = j % _NB
        off = pl.multiple_of(w * _EPW + j * _C, 8)
        il[b] = pltpu.async_copy(idx_hbm.at[pl.ds(off, _C)], ibufs[b], isems[b])
        ml[b] = pltpu.async_copy(msgs_hbm.at[pl.ds(off, _C)], mbufs[b], msems[b])

    for j in range(min(_NB - 1, _CH)):
        start_loads(j)
    for j in range(_CH):
        b = j % _NB
        il[b].wait()
        ml[b].wait()
        pltpu.sync_copy(mbufs[b], aggsh.at[ibufs[b]], add=True)
        if j + _NB - 1 < _CH:
            start_loads(j + _NB - 1)

    plsc.subcore_barrier()
    doff = pl.multiple_of(s * _NPS, 8)
    pltpu.sync_copy(aggsh.at[pl.ds(doff, _NPS)],
                    out_hbm.at[c, pl.ds(doff, _NPS)])


@functools.partial(
    pl.kernel,
    out_type=jax.ShapeDtypeStruct((_P,), jnp.float32),
    mesh=_mesh,
    scratch_types=[
        pltpu.VMEM((_N,), jnp.float32),
        pltpu.VMEM((_PPW,), jnp.int32),
        pltpu.VMEM((_PPW,), jnp.int32),
        pltpu.VMEM((_PPW,), jnp.float32),
    ],
    compiler_params=pltpu.CompilerParams(needs_layout_passes=False),
)
def _sc_pref(util_hbm, ia_hbm, ib_hbm, out_hbm, utilv, iav, ibv, outv):
    w = _worker_id()
    base = pl.multiple_of(w * _PPW, 8)
    pltpu.sync_copy(util_hbm, utilv)
    pltpu.sync_copy(ia_hbm.at[pl.ds(base, _PPW)], iav)
    pltpu.sync_copy(ib_hbm.at[pl.ds(base, _PPW)], ibv)

    def body(j, carry):
        ia = iav[pl.ds(j * 16, 16)]
        ib = ibv[pl.ds(j * 16, 16)]
        va = plsc.load_gather(utilv, [ia])
        vb = plsc.load_gather(utilv, [ib])
        outv[pl.ds(j * 16, 16)] = vb - va
        return carry

    lax.fori_loop(0, _PPW // 16, body, 0)
    pltpu.sync_copy(outv, out_hbm.at[pl.ds(base, _PPW)])


_DK = (_DE + 1) * _H


def _tc_combine(xs, e1, w_cat, sel, red, f_in, block):
    def body(xs_ref, e_ref, w_ref, s_ref, r_ref, o_ref):
        g = jnp.dot(xs_ref[...], w_ref[...], preferred_element_type=jnp.float32)
        ee = jnp.dot(e_ref[...], s_ref[...], preferred_element_type=jnp.float32)
        msgs = jnp.dot(g * ee, r_ref[...], preferred_element_type=jnp.float32)
        o_ref[:, :_H] = msgs
        o_ref[:, _H:] = jnp.zeros((block, _F - _H), jnp.float32)

    nblk = _EPAD // block
    return pl.pallas_call(
        body,
        grid=(nblk,),
        in_specs=[
            pl.BlockSpec((block, f_in), lambda i: (i, 0)),
            pl.BlockSpec((block, _H), lambda i: (i, 0)),
            pl.BlockSpec((f_in, _DK), lambda i: (0, 0)),
            pl.BlockSpec((_H, _DK), lambda i: (0, 0)),
            pl.BlockSpec((_DK, _H), lambda i: (0, 0)),
        ],
        out_specs=pl.BlockSpec((block, _F), lambda i: (i, 0)),
        out_shape=jax.ShapeDtypeStruct((_EPAD, _F), jnp.float32),
    )(xs, e1, w_cat, sel, red)


def _tc_relu_root(agg, x, w_root, b, f_in, block):
    def body(agg_ref, x_ref, w_ref, b_ref, o_ref):
        a = (agg_ref[0, :, :_H] + agg_ref[1, :, :_H]
             + jnp.dot(x_ref[...], w_ref[...], preferred_element_type=jnp.float32)
             + b_ref[...])
        o_ref[:, :_H] = jnp.maximum(a, 0.0)
        o_ref[:, _H:] = jnp.zeros((block, _F - _H), jnp.float32)

    nblk = _N // block
    return pl.pallas_call(
        body,
        grid=(nblk,),
        in_specs=[
            pl.BlockSpec((2, block, _F), lambda i: (0, i, 0)),
            pl.BlockSpec((block, f_in), lambda i: (i, 0)),
            pl.BlockSpec((f_in, _H), lambda i: (0, 0)),
            pl.BlockSpec((1, _H), lambda i: (0, 0)),
        ],
        out_specs=pl.BlockSpec((block, _F), lambda i: (i, 0)),
        out_shape=jax.ShapeDtypeStruct((_N, _F), jnp.float32),
    )(agg, x, w_root, b)


def _tc_final(agg, x1, w_root, b, wd_row, bd, block):
    def body(agg_ref, x_ref, w_ref, b_ref, wd_ref, bd_ref, o_ref):
        a = (agg_ref[0, :, :_H] + agg_ref[1, :, :_H]
             + jnp.dot(x_ref[...], w_ref[...], preferred_element_type=jnp.float32)
             + b_ref[...])
        x2 = jnp.maximum(a, 0.0)
        o_ref[...] = jnp.sum(x2 * wd_ref[...], axis=1, keepdims=True) + bd_ref[...]

    nblk = _N // block
    return pl.pallas_call(
        body,
        grid=(nblk,),
        in_specs=[
            pl.BlockSpec((2, block, _F), lambda i: (0, i, 0)),
            pl.BlockSpec((block, _F), lambda i: (i, 0)),
            pl.BlockSpec((_F, _H), lambda i: (0, 0)),
            pl.BlockSpec((1, _H), lambda i: (0, 0)),
            pl.BlockSpec((1, _H), lambda i: (0, 0)),
            pl.BlockSpec((1, 1), lambda i: (0, 0)),
        ],
        out_specs=pl.BlockSpec((block, 1), lambda i: (i, 0)),
        out_shape=jax.ShapeDtypeStruct((_N, 1), jnp.float32),
    )(agg, x1, w_root, b, wd_row, bd)


def kernel(x, edge_index, e, i, idx_a, idx_b, W_e1, b_e1, W_root1, b1,
           W_e2, b_e2, W_root2, b2, W_d, b_d):
    del i
    x = x.astype(jnp.float32)
    npad = _EPAD - _E
    src = jnp.concatenate(
        [edge_index[0], jnp.zeros((npad,), jnp.int32)]).reshape(_NW, _CH, _C)
    dst_flat = jnp.concatenate(
        [edge_index[1], jnp.full((npad,), _N, jnp.int32)])
    e1 = jnp.concatenate(
        [e, jnp.zeros((npad, _DE), jnp.float32)])
    e1 = jnp.concatenate(
        [e1, jnp.ones((_EPAD, 1), jnp.float32),
         jnp.zeros((_EPAD, _H - _DE - 1), jnp.float32)], axis=1)
    cols = jnp.arange(_DK, dtype=jnp.int32)
    rows = jnp.arange(_H, dtype=jnp.int32)
    sel = (cols[None, :] // _H == rows[:, None]).astype(jnp.float32)
    red = (cols[:, None] % _H == rows[None, :]).astype(jnp.float32)

    w1 = W_e1.reshape(_DE, _F, _H).transpose(1, 0, 2).reshape(_F, _DE * _H)
    w1_cat = jnp.concatenate([w1, b_e1.reshape(_F, _H)], axis=1)
    w2 = W_e2.reshape(_DE, _H, _H).transpose(1, 0, 2).reshape(_H, _DE * _H)
    w2_cat = jnp.pad(jnp.concatenate([w2, b_e2.reshape(_H, _H)], axis=1),
                     ((0, _F - _H), (0, 0)))
    w_root2 = jnp.pad(W_root2, ((0, _F - _H), (0, 0)))

    xs1 = _sc_gather(x, src)
    msgs1 = _tc_combine(xs1, e1, w1_cat, sel, red, _F, 2048)
    agg1 = _sc_scatter_add(msgs1, dst_flat)
    x1 = _tc_relu_root(agg1, x, W_root1, b1.reshape(1, _H), _F, 2000)

    xs2 = _sc_gather(x1, src)
    msgs2 = _tc_combine(xs2, e1, w2_cat, sel, red, _F, 2048)
    agg2 = _sc_scatter_add(msgs2, dst_flat)
    util = _tc_final(agg2, x1, w_root2, b2.reshape(1, _H),
                     W_d.reshape(1, _H), b_d.reshape(1, 1), 2000)

    out = _sc_pref(util.reshape(_N), idx_a, idx_b)
    return out.reshape(_P, 1)


# speedup vs baseline: 5.6933x; 1.1060x over previous
import functools

import jax
import jax.numpy as jnp
from jax import lax
from jax.experimental import pallas as pl
from jax.experimental.pallas import tpu as pltpu
from jax.experimental.pallas import tpu_sc as plsc

_N = 10000
_E = 160000
_F = 128
_H = 32
_DE = 16
_P = 8192

_NC = 2
_NS = 16
_NW = _NC * _NS
_C = 128
_CH = 40
_EPW = _CH * _C
_EPAD = _NW * _EPW
_NA = 10240
_NPS = _NA // _NS
_ZB = 64
_PPW = _P // _NW
_NB = 2

_mesh = plsc.VectorSubcoreMesh(core_axis_name="c", subcore_axis_name="s")


def _worker_id():
    return lax.axis_index("s") * _NC + lax.axis_index("c")


@functools.partial(
    pl.kernel,
    out_type=jax.ShapeDtypeStruct((_EPAD, _F), jnp.float32),
    mesh=_mesh,
    scratch_types=(
        [pltpu.VMEM((_CH, _C), jnp.int32),
         pltpu.VMEM_SHARED((_NA, _F), jnp.float32)]
        + [pltpu.VMEM((_C, _F), jnp.float32) for _ in range(_NB)]
        + [pltpu.SemaphoreType.DMA for _ in range(2 * _NB)]
    ),
)
def _sc_gather(table_hbm, idx_hbm, out_hbm, idxv, tsh,
               r0, r1, g0, g1, q0, q1):
    s = lax.axis_index("s")
    w = _worker_id()
    bufs = [r0, r1]
    gsems = [g0, g1]
    wsems = [q0, q1]

    stripe = _NA // _NS
    soff = pl.multiple_of(s * stripe, 8)

    @pl.when(s < _NS - 1)
    def _():
        pltpu.sync_copy(table_hbm.at[pl.ds(soff, stripe)],
                        tsh.at[pl.ds(soff, stripe)])

    @pl.when(s == _NS - 1)
    def _():
        last = _N - (_NS - 1) * stripe
        loff = pl.multiple_of((_NS - 1) * stripe, 8)
        pltpu.sync_copy(table_hbm.at[pl.ds(loff, last)],
                        tsh.at[pl.ds(loff, last)])

    pltpu.sync_copy(idx_hbm.at[w], idxv)
    plsc.subcore_barrier()

    g = [None] * _NB
    wr = [None] * _NB
    for j in range(min(_NB - 1, _CH)):
        g[j] = pltpu.async_copy(tsh.at[idxv.at[j]], bufs[j], gsems[j])
    for j in range(_CH):
        b = j % _NB
        g[b].wait()
        off = pl.multiple_of(w * _EPW + j * _C, 8)
        wr[b] = pltpu.async_copy(bufs[b], out_hbm.at[pl.ds(off, _C)], wsems[b])
        nx = j + _NB - 1
        if nx < _CH:
            nb = nx % _NB
            if wr[nb] is not None:
                wr[nb].wait()
            g[nb] = pltpu.async_copy(tsh.at[idxv.at[nx]], bufs[nb], gsems[nb])
    for b in range(_NB):
        if wr[b] is not None:
            wr[b].wait()


@functools.partial(
    pl.kernel,
    out_type=jax.ShapeDtypeStruct((_NC, _NA, _F), jnp.float32),
    mesh=_mesh,
    scratch_types=(
        [pltpu.VMEM((_ZB, _F), jnp.float32),
         pltpu.VMEM_SHARED((_NA, _F), jnp.float32)]
        + [pltpu.VMEM((_C,), jnp.int32) for _ in range(_NB)]
        + [pltpu.VMEM((_C, _F), jnp.float32) for _ in range(_NB)]
        + [pltpu.SemaphoreType.DMA for _ in range(3 * _NB)]
    ),
)
def _sc_scatter_add(msgs_hbm, idx_hbm, out_hbm, zbuf, aggsh,
                    i0, i1, m0, m1, s0, s1, t0, t1, u0, u1):
    c = lax.axis_index("c")
    s = lax.axis_index("s")
    w = _worker_id()
    ibufs = [i0, i1]
    mbufs = [m0, m1]
    isems = [s0, s1]
    msems = [t0, t1]
    ssems = [u0, u1]

    def zfill(i, carry):
        for q in range(_F // 16):
            zbuf[i, pl.ds(q * 16, 16)] = jnp.zeros((16,), jnp.float32)
        return carry

    lax.fori_loop(0, _ZB, zfill, 0)
    for k in range(_NPS // _ZB):
        zoff = pl.multiple_of(s * _NPS + k * _ZB, 8)
        pltpu.sync_copy(zbuf, aggsh.at[pl.ds(zoff, _ZB)])
    plsc.subcore_barrier()

    il = [None] * _NB
    ml = [None] * _NB
    sl = [None] * _NB

    def start_loads(j):
        b = j % _NB
        off = pl.multiple_of(w * _EPW + j * _C, 8)
        il[b] = pltpu.async_copy(idx_hbm.at[pl.ds(off, _C)], ibufs[b], isems[b])
        ml[b] = pltpu.async_copy(msgs_hbm.at[pl.ds(off, _C)], mbufs[b], msems[b])

    for j in range(min(_NB - 1, _CH)):
        start_loads(j)
    for j in range(_CH):
        b = j % _NB
        il[b].wait()
        ml[b].wait()
        sl[b] = pltpu.async_copy(mbufs[b], aggsh.at[ibufs[b]], ssems[b],
                                 add=True)
        nx = j + _NB - 1
        if nx < _CH:
            nb = nx % _NB
            if sl[nb] is not None:
                sl[nb].wait()
            start_loads(nx)
    for b in range(_NB):
        if sl[b] is not None:
            sl[b].wait()

    plsc.subcore_barrier()
    doff = pl.multiple_of(s * _NPS, 8)
    pltpu.sync_copy(aggsh.at[pl.ds(doff, _NPS)],
                    out_hbm.at[c, pl.ds(doff, _NPS)])


@functools.partial(
    pl.kernel,
    out_type=jax.ShapeDtypeStruct((_P,), jnp.float32),
    mesh=_mesh,
    scratch_types=[
        pltpu.VMEM((_N,), jnp.float32),
        pltpu.VMEM((_PPW,), jnp.int32),
        pltpu.VMEM((_PPW,), jnp.int32),
        pltpu.VMEM((_PPW,), jnp.float32),
    ],
    compiler_params=pltpu.CompilerParams(needs_layout_passes=False),
)
def _sc_pref(util_hbm, ia_hbm, ib_hbm, out_hbm, utilv, iav, ibv, outv):
    w = _worker_id()
    base = pl.multiple_of(w * _PPW, 8)
    pltpu.sync_copy(util_hbm, utilv)
    pltpu.sync_copy(ia_hbm.at[pl.ds(base, _PPW)], iav)
    pltpu.sync_copy(ib_hbm.at[pl.ds(base, _PPW)], ibv)

    def body(j, carry):
        ia = iav[pl.ds(j * 16, 16)]
        ib = ibv[pl.ds(j * 16, 16)]
        va = plsc.load_gather(utilv, [ia])
        vb = plsc.load_gather(utilv, [ib])
        outv[pl.ds(j * 16, 16)] = vb - va
        return carry

    lax.fori_loop(0, _PPW // 16, body, 0)
    pltpu.sync_copy(outv, out_hbm.at[pl.ds(base, _PPW)])


_DK = (_DE + 1) * _H


def _tc_combine(xs, e1, w_cat, sel, red, f_in, block):
    def body(xs_ref, e_ref, w_ref, s_ref, r_ref, o_ref):
        g = jnp.dot(xs_ref[...], w_ref[...], preferred_element_type=jnp.float32)
        ee = jnp.dot(e_ref[...], s_ref[...], preferred_element_type=jnp.float32)
        msgs = jnp.dot(g * ee, r_ref[...], preferred_element_type=jnp.float32)
        o_ref[:, :_H] = msgs
        o_ref[:, _H:] = jnp.zeros((block, _F - _H), jnp.float32)

    nblk = _EPAD // block
    return pl.pallas_call(
        body,
        grid=(nblk,),
        in_specs=[
            pl.BlockSpec((block, f_in), lambda i: (i, 0)),
            pl.BlockSpec((block, _H), lambda i: (i, 0)),
            pl.BlockSpec((f_in, _DK), lambda i: (0, 0)),
            pl.BlockSpec((_H, _DK), lambda i: (0, 0)),
            pl.BlockSpec((_DK, _H), lambda i: (0, 0)),
        ],
        out_specs=pl.BlockSpec((block, _F), lambda i: (i, 0)),
        out_shape=jax.ShapeDtypeStruct((_EPAD, _F), jnp.float32),
    )(xs, e1, w_cat, sel, red)


def _tc_relu_root(agg, x, w_root, b, f_in, block):
    def body(agg_ref, x_ref, w_ref, b_ref, o_ref):
        a = (agg_ref[0, :, :_H] + agg_ref[1, :, :_H]
             + jnp.dot(x_ref[...], w_ref[...], preferred_element_type=jnp.float32)
             + b_ref[...])
        o_ref[:, :_H] = jnp.maximum(a, 0.0)
        o_ref[:, _H:] = jnp.zeros((block, _F - _H), jnp.float32)

    nblk = _N // block
    return pl.pallas_call(
        body,
        grid=(nblk,),
        in_specs=[
            pl.BlockSpec((2, block, _F), lambda i: (0, i, 0)),
            pl.BlockSpec((block, f_in), lambda i: (i, 0)),
            pl.BlockSpec((f_in, _H), lambda i: (0, 0)),
            pl.BlockSpec((1, _H), lambda i: (0, 0)),
        ],
        out_specs=pl.BlockSpec((block, _F), lambda i: (i, 0)),
        out_shape=jax.ShapeDtypeStruct((_N, _F), jnp.float32),
    )(agg, x, w_root, b)


def _tc_final(agg, x1, w_root, b, wd_row, bd, block):
    def body(agg_ref, x_ref, w_ref, b_ref, wd_ref, bd_ref, o_ref):
        a = (agg_ref[0, :, :_H] + agg_ref[1, :, :_H]
             + jnp.dot(x_ref[...], w_ref[...], preferred_element_type=jnp.float32)
             + b_ref[...])
        x2 = jnp.maximum(a, 0.0)
        o_ref[...] = jnp.sum(x2 * wd_ref[...], axis=1, keepdims=True) + bd_ref[...]

    nblk = _N // block
    return pl.pallas_call(
        body,
        grid=(nblk,),
        in_specs=[
            pl.BlockSpec((2, block, _F), lambda i: (0, i, 0)),
            pl.BlockSpec((block, _F), lambda i: (i, 0)),
            pl.BlockSpec((_F, _H), lambda i: (0, 0)),
            pl.BlockSpec((1, _H), lambda i: (0, 0)),
            pl.BlockSpec((1, _H), lambda i: (0, 0)),
            pl.BlockSpec((1, 1), lambda i: (0, 0)),
        ],
        out_specs=pl.BlockSpec((block, 1), lambda i: (i, 0)),
        out_shape=jax.ShapeDtypeStruct((_N, 1), jnp.float32),
    )(agg, x1, w_root, b, wd_row, bd)


def kernel(x, edge_index, e, i, idx_a, idx_b, W_e1, b_e1, W_root1, b1,
           W_e2, b_e2, W_root2, b2, W_d, b_d):
    del i
    x = x.astype(jnp.float32)
    npad = _EPAD - _E
    src = jnp.concatenate(
        [edge_index[0], jnp.zeros((npad,), jnp.int32)]).reshape(_NW, _CH, _C)
    dst_flat = jnp.concatenate(
        [edge_index[1], jnp.full((npad,), _N, jnp.int32)])
    e1 = jnp.concatenate(
        [e, jnp.zeros((npad, _DE), jnp.float32)])
    e1 = jnp.concatenate(
        [e1, jnp.ones((_EPAD, 1), jnp.float32),
         jnp.zeros((_EPAD, _H - _DE - 1), jnp.float32)], axis=1)
    cols = jnp.arange(_DK, dtype=jnp.int32)
    rows = jnp.arange(_H, dtype=jnp.int32)
    sel = (cols[None, :] // _H == rows[:, None]).astype(jnp.float32)
    red = (cols[:, None] % _H == rows[None, :]).astype(jnp.float32)

    w1 = W_e1.reshape(_DE, _F, _H).transpose(1, 0, 2).reshape(_F, _DE * _H)
    w1_cat = jnp.concatenate([w1, b_e1.reshape(_F, _H)], axis=1)
    w2 = W_e2.reshape(_DE, _H, _H).transpose(1, 0, 2).reshape(_H, _DE * _H)
    w2_cat = jnp.pad(jnp.concatenate([w2, b_e2.reshape(_H, _H)], axis=1),
                     ((0, _F - _H), (0, 0)))
    w_root2 = jnp.pad(W_root2, ((0, _F - _H), (0, 0)))

    xs1 = _sc_gather(x, src)
    msgs1 = _tc_combine(xs1, e1, w1_cat, sel, red, _F, 4096)
    agg1 = _sc_scatter_add(msgs1, dst_flat)
    x1 = _tc_relu_root(agg1, x, W_root1, b1.reshape(1, _H), _F, 2000)

    xs2 = _sc_gather(x1, src)
    msgs2 = _tc_combine(xs2, e1, w2_cat, sel, red, _F, 4096)
    agg2 = _sc_scatter_add(msgs2, dst_flat)
    util = _tc_final(agg2, x1, w_root2, b2.reshape(1, _H),
                     W_d.reshape(1, _H), b_d.reshape(1, 1), 2000)

    out = _sc_pref(util.reshape(_N), idx_a, idx_b)
    return out.reshape(_P, 1)
